# baseline (speedup 1.0000x reference)
import jax
import jax.numpy as jnp
from jax.experimental import pallas as pl
from jax.experimental.pallas import tpu as pltpu

_BM = 400
_NBUF = 3


def _gcn_body(x_hbm, w_hbm, b_hbm, adj_hbm, out_hbm,
              xv, wv, bv, sup, adjbuf, outbuf,
              adj_sems, out_sems, in_sems):
    n = x_hbm.shape[0]
    nsteps = n // _BM

    cx = pltpu.make_async_copy(x_hbm, xv, in_sems.at[0])
    cw = pltpu.make_async_copy(w_hbm, wv, in_sems.at[1])
    cb = pltpu.make_async_copy(b_hbm, bv, in_sems.at[2])
    cx.start()
    cw.start()
    cb.start()

    def adj_copy(step, slot):
        return pltpu.make_async_copy(
            adj_hbm.at[pl.ds(step * _BM, _BM)],
            adjbuf.at[slot],
            adj_sems.at[slot],
        )

    def out_copy(step, oslot):
        return pltpu.make_async_copy(
            outbuf.at[oslot],
            out_hbm.at[pl.ds(step * _BM, _BM)],
            out_sems.at[oslot],
        )

    for slot in range(_NBUF):
        adj_copy(slot, slot).start()

    cx.wait()
    cw.wait()
    sup[...] = jnp.dot(
        xv[...], wv[...], preferred_element_type=jnp.float32
    ).astype(jnp.bfloat16)
    cb.wait()

    for step in range(nsteps):
        slot = step % _NBUF
        oslot = step % 2
        adj_copy(step, slot).wait()
        if step >= 2:
            out_copy(step - 2, oslot).wait()
        outbuf[oslot] = (
            jnp.dot(adjbuf[slot].astype(jnp.bfloat16), sup[...],
                    preferred_element_type=jnp.float32)
            + bv[...]
        )
        out_copy(step, oslot).start()
        nxt = step + _NBUF
        if nxt < nsteps:
            adj_copy(nxt, slot).start()

    out_copy(nsteps - 2, (nsteps - 2) % 2).wait()
    out_copy(nsteps - 1, (nsteps - 1) % 2).wait()


def kernel(inputs, adj, weight, bias):
    n, d_in = inputs.shape
    d_out = weight.shape[1]
    bias2 = bias.reshape(1, d_out)
    return pl.pallas_call(
        _gcn_body,
        in_specs=[
            pl.BlockSpec(memory_space=pl.ANY),
            pl.BlockSpec(memory_space=pl.ANY),
            pl.BlockSpec(memory_space=pl.ANY),
            pl.BlockSpec(memory_space=pl.ANY),
        ],
        out_specs=pl.BlockSpec(memory_space=pl.ANY),
        out_shape=jax.ShapeDtypeStruct((n, d_out), jnp.float32),
        scratch_shapes=[
            pltpu.VMEM((n, d_in), jnp.float32),
            pltpu.VMEM((d_in, d_out), jnp.float32),
            pltpu.VMEM((1, d_out), jnp.float32),
            pltpu.VMEM((n, d_out), jnp.bfloat16),
            pltpu.VMEM((_NBUF, _BM, n), jnp.float32),
            pltpu.VMEM((2, _BM, d_out), jnp.float32),
            pltpu.SemaphoreType.DMA((_NBUF,)),
            pltpu.SemaphoreType.DMA((2,)),
            pltpu.SemaphoreType.DMA((3,)),
        ],
    )(inputs, weight, bias2, adj)


# speedup vs baseline: 1.0051x; 1.0051x over previous
import jax
import jax.numpy as jnp
from jax.experimental import pallas as pl
from jax.experimental.pallas import tpu as pltpu


def _gcn_body(x_ref, w_ref, b_ref, adj_ref, out_ref, support_ref):
    i = pl.program_id(0)

    @pl.when(i == 0)
    def _():
        support_ref[...] = jnp.dot(
            x_ref[...], w_ref[...], preferred_element_type=jnp.float32
        ).astype(jnp.bfloat16)

    out_ref[...] = (
        jnp.dot(
            adj_ref[...].astype(jnp.bfloat16),
            support_ref[...],
            preferred_element_type=jnp.float32,
        )
        + b_ref[...]
    )


def kernel(inputs, adj, weight, bias):
    n, d_in = inputs.shape
    d_out = weight.shape[1]
    bm = min(416, ((n + 7) // 8) * 8)
    bias2 = bias.reshape(1, d_out)
    return pl.pallas_call(
        _gcn_body,
        grid=(pl.cdiv(n, bm),),
        in_specs=[
            pl.BlockSpec((n, d_in), lambda i: (0, 0)),
            pl.BlockSpec((d_in, d_out), lambda i: (0, 0)),
            pl.BlockSpec((1, d_out), lambda i: (0, 0)),
            pl.BlockSpec((bm, n), lambda i: (i, 0)),
        ],
        out_specs=pl.BlockSpec((bm, d_out), lambda i: (i, 0)),
        out_shape=jax.ShapeDtypeStruct((n, d_out), jnp.float32),
        scratch_shapes=[pltpu.VMEM((n, d_out), jnp.bfloat16)],
    )(inputs, weight, bias2, adj)


# speedup vs baseline: 1.0235x; 1.0183x over previous
import jax
import jax.numpy as jnp
from jax.experimental import pallas as pl
from jax.experimental.pallas import tpu as pltpu


def _gcn_body(x_ref, w_ref, b_ref, adj_ref, out_ref, support_ref):
    i = pl.program_id(0)

    @pl.when(i == 0)
    def _():
        support_ref[...] = jnp.dot(
            x_ref[...], w_ref[...], preferred_element_type=jnp.float32
        ).astype(jnp.bfloat16)

    out_ref[...] = (
        jnp.dot(
            adj_ref[...].astype(jnp.bfloat16),
            support_ref[...],
            preferred_element_type=jnp.float32,
        )
        + b_ref[...]
    )


def kernel(inputs, adj, weight, bias):
    n, d_in = inputs.shape
    d_out = weight.shape[1]
    bm = 400 if n % 400 == 0 else min(400, ((n + 7) // 8) * 8)
    bias2 = bias.reshape(1, d_out)
    return pl.pallas_call(
        _gcn_body,
        grid=(pl.cdiv(n, bm),),
        in_specs=[
            pl.BlockSpec((n, d_in), lambda i: (0, 0)),
            pl.BlockSpec((d_in, d_out), lambda i: (0, 0)),
            pl.BlockSpec((1, d_out), lambda i: (0, 0)),
            pl.BlockSpec((bm, n), lambda i: (i, 0)),
        ],
        out_specs=pl.BlockSpec((bm, d_out), lambda i: (i, 0)),
        out_shape=jax.ShapeDtypeStruct((n, d_out), jnp.float32),
        scratch_shapes=[pltpu.VMEM((n, d_out), jnp.bfloat16)],
    )(inputs, weight, bias2, adj)
